# Initial kernel scaffold; baseline (speedup 1.0000x reference)
import jax
import jax.numpy as jnp
from jax.experimental import pallas as pl


def kernel(p, queue, mask, label):
    raise NotImplementedError("write your pallas kernel here")


# speedup vs baseline: 2.3472x; 2.3472x over previous
import functools

import jax
import jax.numpy as jnp
from jax.experimental import pallas as pl
from jax.experimental.pallas import tpu as pltpu

_Q = 7409
_D = 512
_B = 1024
_SCALE = 32.0
_MARGIN = 0.4
_HARD_NEG = 3
_C = 1024
_NB = -(-_Q // _C)
_NEG_INF = -1e30


def _ffc_body(label_ref, p_ref, q0_ref, q1_ref, mask_ref, out_ref,
              pn, m1, s1, v1, a1, b1, c1s, m2, s2, v2, a2, b2, c2s):
    j = pl.program_id(0)

    @pl.when(j == 0)
    def _init():
        pv = p_ref[...]
        psq = jnp.sum(pv * pv, axis=1, keepdims=True)
        pn[...] = pv * jax.lax.rsqrt(psq)
        neg = jnp.full((_B, 1), _NEG_INF, jnp.float32)
        zero = jnp.zeros((_B, 1), jnp.float32)
        for r in (m1, m2, a1, b1, c1s, a2, b2, c2s):
            r[...] = neg
        for r in (s1, s2, v1, v2):
            r[...] = zero

    q0 = q0_ref[...]
    q1 = q1_ref[...]
    mk = mask_ref[...]
    w = mk * q1 + (1.0 - mk) * q0
    pnv = pn[...]
    dn = (((1,), (1,)), ((), ()))
    cos1 = jax.lax.dot_general(pnv, q0, dn, preferred_element_type=jnp.float32)
    cos2 = jax.lax.dot_general(pnv, w, dn, preferred_element_type=jnp.float32)

    colid = jax.lax.broadcasted_iota(jnp.int32, (_B, _C), 1) + j * _C
    valid = colid < _Q
    lab = label_ref[...]
    safe = jnp.where(lab < 0, 0, lab)
    is_lab = colid == safe

    def update(c, m, s, v, ta, tb, tc):
        cm = jnp.where(valid, c, _NEG_INF)
        bmax = jnp.max(cm, axis=1, keepdims=True)
        mold = m[...]
        mnew = jnp.maximum(mold, _SCALE * bmax)
        ladj = _SCALE * cm - jnp.where(is_lab, _SCALE * _MARGIN, 0.0)
        e = jnp.exp(ladj - mnew)
        s[...] = s[...] * jnp.exp(mold - mnew) + jnp.sum(e, axis=1, keepdims=True)
        m[...] = mnew
        v[...] = v[...] + jnp.sum(jnp.where(is_lab, cm, 0.0), axis=1, keepdims=True)
        t1v, t2v, t3v = ta[...], tb[...], tc[...]
        work = cm
        bm = bmax
        for r in range(3):
            if r > 0:
                bm = jnp.max(work, axis=1, keepdims=True)
            x1 = jnp.minimum(t1v, bm)
            t1v = jnp.maximum(t1v, bm)
            x2 = jnp.minimum(t2v, x1)
            t2v = jnp.maximum(t2v, x1)
            t3v = jnp.maximum(t3v, x2)
            if r < 2:
                ii = jnp.min(jnp.where(work == bm, colid, jnp.int32(1 << 30)),
                             axis=1, keepdims=True)
                work = jnp.where(colid == ii, _NEG_INF, work)
        ta[...] = t1v
        tb[...] = t2v
        tc[...] = t3v

    update(cos1, m1, s1, v1, a1, b1, c1s)
    update(cos2, m2, s2, v2, a2, b2, c2s)

    @pl.when(j == _NB - 1)
    def _final():
        posf = (label_ref[...] >= 0).astype(jnp.float32)
        n_pos = jnp.sum(posf)
        n_neg = jnp.float32(_B) - n_pos
        total = jnp.float32(0.0)
        for (m, s, v, ta, tb, tc) in ((m1, s1, v1, a1, b1, c1s),
                                      (m2, s2, v2, a2, b2, c2s)):
            ce = m[...] + jnp.log(s[...]) - _SCALE * (v[...] - _MARGIN)
            cls = jnp.where(n_pos > 0,
                            jnp.sum(ce * posf) / jnp.maximum(n_pos, 1.0), 0.0)
            hard = (jnp.maximum(ta[...], 0.0) + jnp.maximum(tb[...], 0.0)
                    + jnp.maximum(tc[...], 0.0))
            negl = jnp.where(n_neg > 0,
                             jnp.sum(hard * (1.0 - posf))
                             / jnp.maximum(n_neg * _HARD_NEG, 1.0), 0.0)
            total = total + cls + negl
        out_ref[...] = jnp.reshape(total, (1, 1))


@functools.partial(jax.jit, static_argnames=())
def kernel(p, queue, mask, label):
    label2d = label.astype(jnp.int32).reshape(_B, 1)
    q0 = queue[0]
    q1 = queue[1]
    stat = lambda: pltpu.VMEM((_B, 1), jnp.float32)
    out = pl.pallas_call(
        _ffc_body,
        grid=(_NB,),
        in_specs=[
            pl.BlockSpec((_B, 1), lambda j: (0, 0)),
            pl.BlockSpec((_B, _D), lambda j: (0, 0)),
            pl.BlockSpec((_C, _D), lambda j: (j, 0)),
            pl.BlockSpec((_C, _D), lambda j: (j, 0)),
            pl.BlockSpec((_C, 1), lambda j: (j, 0)),
        ],
        out_specs=pl.BlockSpec((1, 1), lambda j: (0, 0)),
        out_shape=jax.ShapeDtypeStruct((1, 1), jnp.float32),
        scratch_shapes=[pltpu.VMEM((_B, _D), jnp.float32)] + [stat() for _ in range(12)],
        compiler_params=pltpu.CompilerParams(
            dimension_semantics=("arbitrary",)),
    )(label2d, p, q0, q1, mask)
    return out[0, 0]


# speedup vs baseline: 2.8987x; 1.2350x over previous
import functools

import jax
import jax.numpy as jnp
from jax.experimental import pallas as pl
from jax.experimental.pallas import tpu as pltpu

_Q = 7409
_D = 512
_B = 1024
_SCALE = 32.0
_MARGIN = 0.4
_HARD_NEG = 3
_C = 1024
_NB = -(-_Q // _C)
_NPAD = _NB * _C - _Q
_NEG_INF = -1e30


def _ffc_body(label_ref, p_ref, q0_ref, q1_ref, mask_ref, out_ref,
              pn, m1, s1, v1, a1, b1, c1s, m2, s2, v2, a2, b2, c2s):
    j = pl.program_id(0)

    @pl.when(j == 0)
    def _init():
        pv = p_ref[...]
        psq = jnp.sum(pv * pv, axis=1, keepdims=True)
        pn[...] = pv * (_SCALE * jax.lax.rsqrt(psq))
        neg = jnp.full((_B, 1), _NEG_INF, jnp.float32)
        zero = jnp.zeros((_B, 1), jnp.float32)
        for r in (m1, m2, a1, b1, c1s, a2, b2, c2s):
            r[...] = neg
        for r in (s1, s2, v1, v2):
            r[...] = zero

    rowid = jax.lax.broadcasted_iota(jnp.int32, (_C, 1), 0) + j * _C
    rvalid = rowid < _Q
    q0 = jnp.where(rvalid, q0_ref[...], 0.0)
    q1 = jnp.where(rvalid, q1_ref[...], 0.0)
    mk = jnp.where(rvalid, mask_ref[...], 0.0)
    w = q0 + mk * (q1 - q0)
    pnv = pn[...]
    dn = (((1,), (1,)), ((), ()))
    z1 = jax.lax.dot_general(pnv, q0, dn, preferred_element_type=jnp.float32)
    z2 = jax.lax.dot_general(pnv, w, dn, preferred_element_type=jnp.float32)

    colid = jax.lax.broadcasted_iota(jnp.int32, (_B, _C), 1) + j * _C
    lab = label_ref[...]
    safe = jnp.where(lab < 0, 0, lab)
    is_lab = colid == safe

    def update(z, m, s, v, ta, tb, tc):
        bmax = jnp.max(z, axis=1, keepdims=True)
        mold = m[...]
        mnew = jnp.maximum(mold, bmax)
        e = jnp.exp(z - mnew)
        s[...] = s[...] * jnp.exp(mold - mnew) + jnp.sum(e, axis=1, keepdims=True)
        m[...] = mnew
        v[...] = v[...] + jnp.sum(jnp.where(is_lab, z, 0.0), axis=1, keepdims=True)
        t1v, t2v, t3v = ta[...], tb[...], tc[...]
        work = z
        bm = bmax
        for r in range(3):
            x1 = jnp.minimum(t1v, bm)
            t1v = jnp.maximum(t1v, bm)
            x2 = jnp.minimum(t2v, x1)
            t2v = jnp.maximum(t2v, x1)
            t3v = jnp.maximum(t3v, x2)
            if r < 2:
                work = jnp.where(work == bm, _NEG_INF, work)
                bm = jnp.max(work, axis=1, keepdims=True)
        ta[...] = t1v
        tb[...] = t2v
        tc[...] = t3v

    update(z1, m1, s1, v1, a1, b1, c1s)
    update(z2, m2, s2, v2, a2, b2, c2s)

    @pl.when(j == _NB - 1)
    def _final():
        posf = (label_ref[...] >= 0).astype(jnp.float32)
        n_pos = jnp.sum(posf)
        n_neg = jnp.float32(_B) - n_pos
        sm = jnp.float32(_SCALE * _MARGIN)
        total = jnp.float32(0.0)
        for (m, s, v, ta, tb, tc) in ((m1, s1, v1, a1, b1, c1s),
                                      (m2, s2, v2, a2, b2, c2s)):
            mv, sv, vv = m[...], s[...], v[...]
            sadj = (sv - jnp.float32(_NPAD) * jnp.exp(-mv)
                    - jnp.exp(vv - mv) + jnp.exp(vv - sm - mv))
            ce = mv + jnp.log(sadj) - vv + sm
            cls = jnp.where(n_pos > 0,
                            jnp.sum(ce * posf) / jnp.maximum(n_pos, 1.0), 0.0)
            hard = (jnp.maximum(ta[...], 0.0) + jnp.maximum(tb[...], 0.0)
                    + jnp.maximum(tc[...], 0.0)) * jnp.float32(1.0 / _SCALE)
            negl = jnp.where(n_neg > 0,
                             jnp.sum(hard * (1.0 - posf))
                             / jnp.maximum(n_neg * _HARD_NEG, 1.0), 0.0)
            total = total + cls + negl
        out_ref[...] = jnp.reshape(total, (1, 1))


@functools.partial(jax.jit, static_argnames=())
def kernel(p, queue, mask, label):
    label2d = label.astype(jnp.int32).reshape(_B, 1)
    q0 = queue[0]
    q1 = queue[1]
    stat = lambda: pltpu.VMEM((_B, 1), jnp.float32)
    out = pl.pallas_call(
        _ffc_body,
        grid=(_NB,),
        in_specs=[
            pl.BlockSpec((_B, 1), lambda j: (0, 0)),
            pl.BlockSpec((_B, _D), lambda j: (0, 0)),
            pl.BlockSpec((_C, _D), lambda j: (j, 0)),
            pl.BlockSpec((_C, _D), lambda j: (j, 0)),
            pl.BlockSpec((_C, 1), lambda j: (j, 0)),
        ],
        out_specs=pl.BlockSpec((1, 1), lambda j: (0, 0)),
        out_shape=jax.ShapeDtypeStruct((1, 1), jnp.float32),
        scratch_shapes=[pltpu.VMEM((_B, _D), jnp.float32)] + [stat() for _ in range(12)],
        compiler_params=pltpu.CompilerParams(
            dimension_semantics=("arbitrary",)),
    )(label2d, p, q0, q1, mask)
    return out[0, 0]
